# baseline (speedup 1.0000x reference)
import functools

import jax
import jax.numpy as jnp
from jax import lax
from jax.experimental import pallas as pl
from jax.experimental.pallas import tpu as pltpu
from jax.experimental.pallas import tpu_sc as plsc

_N = 4194304
_BLK_TC = 262144
_GRID = _N // _BLK_TC

_NC = 2
_NS = 16
_NW = _NC * _NS
_CHUNK = _N // _NW
_BLK = 16384
_NBLK = _CHUNK // _BLK
_LANES = 16


def _tc_body(tab_ref, idx_ref, out_ref):
    i = idx_ref[...]
    t0 = tab_ref[0]
    t1 = tab_ref[1]
    t2 = tab_ref[2]
    t3 = tab_ref[3]
    lo = jnp.where(i == 0, t0, t1)
    hi = jnp.where(i == 2, t2, t3)
    out_ref[...] = jnp.where(i < 2, lo, hi)


_tc_gather = pl.pallas_call(
    _tc_body,
    grid=(_GRID,),
    in_specs=[
        pl.BlockSpec(memory_space=pltpu.SMEM),
        pl.BlockSpec((_BLK_TC,), lambda i: (i,)),
    ],
    out_specs=pl.BlockSpec((_BLK_TC,), lambda i: (i,)),
    out_shape=jax.ShapeDtypeStruct((_N,), jnp.float32),
    compiler_params=pltpu.CompilerParams(
        dimension_semantics=("arbitrary",),
    ),
)

_mesh = plsc.VectorSubcoreMesh(core_axis_name="c", subcore_axis_name="s")


@functools.partial(
    pl.kernel,
    mesh=_mesh,
    out_type=jax.ShapeDtypeStruct((_N,), jnp.float32),
    scratch_types=[
        pltpu.VMEM((_LANES,), jnp.float32),
        pltpu.VMEM((_BLK,), jnp.int32),
        pltpu.VMEM((_BLK,), jnp.int32),
        pltpu.VMEM((_BLK,), jnp.float32),
        pltpu.VMEM((_BLK,), jnp.float32),
        pltpu.SemaphoreType.DMA,
        pltpu.SemaphoreType.DMA,
        pltpu.SemaphoreType.DMA,
        pltpu.SemaphoreType.DMA,
    ],
)
def _gather_sc(table_hbm, idx_hbm, out_hbm, table_v,
               idx_v0, idx_v1, out_v0, out_v1,
               in_s0, in_s1, out_s0, out_s1):
    wid = lax.axis_index("s") * _NC + lax.axis_index("c")
    base = wid * _CHUNK
    pltpu.sync_copy(table_hbm, table_v)
    tab = table_v[...]

    idx_bufs = (idx_v0, idx_v1)
    out_bufs = (out_v0, out_v1)
    in_sems = (in_s0, in_s1)
    out_sems = (out_s0, out_s1)

    in_copies = {}
    out_copies = {}

    def start_in(b):
        in_copies[b] = pltpu.async_copy(
            idx_hbm.at[pl.ds(base + b * _BLK, _BLK)],
            idx_bufs[b % 2], in_sems[b % 2])

    start_in(0)
    for b in range(_NBLK):
        ib = idx_bufs[b % 2]
        ob = out_bufs[b % 2]
        in_copies.pop(b).wait()
        if b + 1 < _NBLK:
            start_in(b + 1)
        if b >= 2:
            out_copies.pop(b - 2).wait()

        @plsc.parallel_loop(0, _BLK, _LANES, unroll=8)
        def body(s):
            s = pl.multiple_of(s, _LANES)
            idx = ib[pl.ds(s, _LANES)]
            ob[pl.ds(s, _LANES)] = tab.at[idx].get(mode="promise_in_bounds")

        out_copies[b] = pltpu.async_copy(
            ob, out_hbm.at[pl.ds(base + b * _BLK, _BLK)], out_sems[b % 2])

    out_copies.pop(_NBLK - 2).wait()
    out_copies.pop(_NBLK - 1).wait()


def kernel(supervision_weight, index, dummy, bin_num_examples):
    table = jnp.pad(bin_num_examples, (0, _LANES - bin_num_examples.shape[0]))
    sc_out = _gather_sc(table, index)
    tc_out = _tc_gather(bin_num_examples, index)
    tc_out, _ = lax.optimization_barrier((tc_out, sc_out))
    return tc_out
